# baseline (speedup 1.0000x reference)
import functools

import jax
import jax.numpy as jnp
from jax import lax
from jax.experimental import pallas as pl
from jax.experimental.pallas import tpu as pltpu
from jax.experimental.pallas import tpu_sc as plsc

NUSERS = 25000
NITEMS = 25000
NN = NUSERS + NITEMS
NP = 50048
D = 64
H = 32
NLAYERS = 3
NEDGES = 800000
GB = 256
NSLOT = 3
NU = 198
NSI = NU // NSLOT
EPT = NU * GB
EPAD = 16 * EPT
RPT = NP // 16
ZC = 136
NZ = RPT // ZC
B = 4096
BPW = B // 32

_mesh = plsc.VectorSubcoreMesh(core_axis_name="c", subcore_axis_name="s")
_params = pltpu.CompilerParams(use_tc_tiling_on_sc=False,
                               needs_layout_passes=False)


def _prop_body(t0_hbm, pk_hbm, ot_hbm,
               acc, pb0, pb1, pb2, rows0, rows1, rows2,
               es0, es1, es2, gs0, gs1, gs2, ss0, ss1, ss2):
    pbs = (pb0, pb1, pb2)
    rws = (rows0, rows1, rows2)
    esem = (es0, es1, es2)
    gsem = (gs0, gs1, gs2)
    ssem = (ss0, ss1, ss2)
    c = lax.axis_index("c")
    w = lax.axis_index("s")
    zv = jnp.zeros((16,), jnp.float32)
    z_dma = rows0.at[pl.ds(0, ZC)]
    b_dma = rows1.at[pl.ds(0, ZC)]

    def zero_head(i, _):
        rows0[i, pl.ds(0, 16)] = zv
        rows0[i, pl.ds(16, 16)] = zv
        return 0

    lax.fori_loop(0, ZC, zero_head, 0)

    def pro(j, _):
        r0 = w * RPT + j * ZC
        pltpu.sync_copy(t0_hbm.at[pl.ds(c * NP + r0, ZC)], b_dma)
        pltpu.sync_copy(b_dma, ot_hbm.at[pl.ds(c * NP + r0, ZC)])
        pltpu.sync_copy(z_dma, acc.at[pl.ds(r0, ZC)])
        return 0
    lax.fori_loop(0, NZ, pro, 0)
    plsc.subcore_barrier()

    ubase = w * NU

    def layer(l, _):
        g_off = (2 * l + c) * NP

        def addoff(pb):
            def ao(i, _):
                for q in range(4):
                    sl = pl.ds(i * 64 + q * 16, 16)
                    pb[0, sl] = pb[0, sl] + g_off
                return 0
            lax.fori_loop(0, GB // 64, ao, 0)

        def scale(pb, rows):
            def s16(t, _):
                sa = pl.ds(0, 16)
                sb = pl.ds(16, 16)
                for q in range(2):
                    vv = plsc.bitcast(
                        pb[2, pl.ds(t * 32 + q * 16, 16)], jnp.float32)
                    for k in range(16):
                        j = t * 32 + q * 16 + k
                        v = vv[k]
                        rows[j, sa] = rows[j, sa] * v
                        rows[j, sb] = rows[j, sb] * v
                return 0
            lax.fori_loop(0, GB // 32, s16, 0)

        pltpu.async_copy(pk_hbm.at[ubase], pb0, es0)
        pltpu.async_copy(pk_hbm.at[ubase + 1], pb1, es1)
        pltpu.make_async_copy(pk_hbm.at[ubase], pb0, es0).wait()
        addoff(pb0)
        pltpu.async_copy(ot_hbm.at[pb0.at[0]], rows0, gs0)

        def step(si, b):
            u = si * NSLOT + b
            bn = (b + 1) % NSLOT
            bp = (b + 2) % NSLOT
            pltpu.make_async_copy(ot_hbm.at[pbs[b].at[0]], rws[b],
                                  gsem[b]).wait()

            def fire_next_gather():
                pltpu.make_async_copy(pk_hbm.at[ubase], pbs[bn],
                                      esem[bn]).wait()
                addoff(pbs[bn])
                pltpu.async_copy(ot_hbm.at[pbs[bn].at[0]], rws[bn],
                                 gsem[bn])
            if b == NSLOT - 1:
                pl.when(si < NSI - 1)(fire_next_gather)
            else:
                fire_next_gather()

            scale(pbs[b], rws[b])
            pltpu.async_copy(rws[b], acc.at[pbs[b].at[1]], ssem[b],
                             add=True)

            def wait_prev_scatter():
                pltpu.make_async_copy(rws[bp], acc.at[pbs[bp].at[1]],
                                      ssem[bp]).wait()
            if b == 0:
                pl.when(si > 0)(wait_prev_scatter)
            else:
                wait_prev_scatter()

            def fire_e():
                pltpu.async_copy(pk_hbm.at[ubase + u + 2], pbs[bp],
                                 esem[bp])
            if b == 0:
                fire_e()
            else:
                pl.when(si < NSI - 1)(fire_e)

        def si_loop(si, _):
            for b in range(NSLOT):
                step(si, b)
            return 0
        lax.fori_loop(0, NSI, si_loop, 0)
        pltpu.make_async_copy(rws[NSLOT - 1],
                              acc.at[pbs[NSLOT - 1].at[1]],
                              ssem[NSLOT - 1]).wait()
        plsc.subcore_barrier()

        lax.fori_loop(0, ZC, zero_head, 0)
        o_off = (2 * (l + 1) + c) * NP

        def cz(j, _):
            r0 = w * RPT + j * ZC
            pltpu.sync_copy(acc.at[pl.ds(r0, ZC)], b_dma)
            pltpu.sync_copy(b_dma, ot_hbm.at[pl.ds(o_off + r0, ZC)])
            pltpu.sync_copy(z_dma, acc.at[pl.ds(r0, ZC)])
            return 0
        lax.fori_loop(0, NZ, cz, 0)
        plsc.subcore_barrier()
        return 0
    lax.fori_loop(0, NLAYERS, layer, 0)

    m_off = (8 + c) * NP

    def mean(j, _):
        r0 = w * RPT + j * ZC
        pltpu.sync_copy(ot_hbm.at[pl.ds(c * NP + r0, ZC)], b_dma)
        for l in range(1, 4):
            pltpu.sync_copy(ot_hbm.at[pl.ds((2 * l + c) * NP + r0, ZC)],
                            z_dma)

            def macc(r, _):
                sa = pl.ds(0, 16)
                sb = pl.ds(16, 16)
                rows1[r, sa] = rows1[r, sa] + rows0[r, sa]
                rows1[r, sb] = rows1[r, sb] + rows0[r, sb]
                return 0
            lax.fori_loop(0, ZC, macc, 0)

        def mscale(r, _):
            sa = pl.ds(0, 16)
            sb = pl.ds(16, 16)
            rows1[r, sa] = rows1[r, sa] * 0.25
            rows1[r, sb] = rows1[r, sb] * 0.25
            return 0
        lax.fori_loop(0, ZC, mscale, 0)
        pltpu.sync_copy(b_dma, ot_hbm.at[pl.ds(m_off + r0, ZC)])
        return 0
    lax.fori_loop(0, NZ, mean, 0)


_prop_kernel = functools.partial(
    pl.kernel,
    out_type=jax.ShapeDtypeStruct((10 * NP, H), jnp.float32),
    mesh=_mesh,
    scratch_types=[
        pltpu.VMEM_SHARED((NP, H), jnp.float32),
        pltpu.VMEM((3, GB), jnp.int32),
        pltpu.VMEM((3, GB), jnp.int32),
        pltpu.VMEM((3, GB), jnp.int32),
        pltpu.VMEM((GB, H), jnp.float32),
        pltpu.VMEM((GB, H), jnp.float32),
        pltpu.VMEM((GB, H), jnp.float32),
        pltpu.SemaphoreType.DMA,
        pltpu.SemaphoreType.DMA,
        pltpu.SemaphoreType.DMA,
        pltpu.SemaphoreType.DMA,
        pltpu.SemaphoreType.DMA,
        pltpu.SemaphoreType.DMA,
        pltpu.SemaphoreType.DMA,
        pltpu.SemaphoreType.DMA,
        pltpu.SemaphoreType.DMA,
    ],
    compiler_params=_params,
)(_prop_body)


def _score_body(ot_hbm, iu_hbm, ii_hbm, ij_hbm, out_i, out_j, out_r,
                b_iu, b_ii, b_ij, gidx, g,
                ulo, uhi, ilo, ihi, jlo, jhi, pi_buf, pj_buf, rbuf, sem):
    c = lax.axis_index("c")
    s = lax.axis_index("s")
    wid = s * 2 + c
    b0 = wid * BPW
    pltpu.sync_copy(iu_hbm.at[pl.ds(b0, BPW)], b_iu)
    pltpu.sync_copy(ii_hbm.at[pl.ds(b0, BPW)], b_ii)
    pltpu.sync_copy(ij_hbm.at[pl.ds(b0, BPW)], b_ij)

    def gather_to(idxbuf, off, dstbuf):
        def addoff(i, _):
            sl = pl.ds(i * 16, 16)
            gidx[sl] = idxbuf[sl] + off
            return 0
        lax.fori_loop(0, BPW // 16, addoff, 0)
        pltpu.async_copy(ot_hbm.at[gidx], dstbuf, sem).wait()

    gather_to(b_iu, 8 * NP, ulo)
    gather_to(b_iu, 9 * NP, uhi)
    gather_to(b_ii, 8 * NP, ilo)
    gather_to(b_ii, 9 * NP, ihi)
    gather_to(b_ij, 8 * NP, jlo)
    gather_to(b_ij, 9 * NP, jhi)

    racc = jnp.zeros((16,), jnp.float32)
    for idxbuf in (b_iu, b_ii, b_ij):
        for h in (0, 1):
            gather_to(idxbuf, h * NP, g)

            def sq(r, a):
                va = g[r, pl.ds(0, 16)]
                vb = g[r, pl.ds(16, 16)]
                return a + va * va + vb * vb
            racc = lax.fori_loop(0, BPW, sq, racc)
    rbuf[pl.ds(0, 16)] = racc
    pltpu.sync_copy(rbuf, out_r.at[wid])

    lanes = lax.iota(jnp.int32, 16)

    def dot16(t, _):
        s0 = pl.ds(0, 16)
        s1 = pl.ds(16, 16)
        piv = jnp.zeros((16,), jnp.float32)
        pjv = jnp.zeros((16,), jnp.float32)
        for k in range(16):
            r = t * 16 + k
            u0 = ulo[r, s0]
            u1 = ulo[r, s1]
            u2 = uhi[r, s0]
            u3 = uhi[r, s1]
            pi = jnp.sum(u0 * ilo[r, s0] + u1 * ilo[r, s1]
                         + u2 * ihi[r, s0] + u3 * ihi[r, s1])
            pj = jnp.sum(u0 * jlo[r, s0] + u1 * jlo[r, s1]
                         + u2 * jhi[r, s0] + u3 * jhi[r, s1])
            piv = jnp.where(lanes == k, pi, piv)
            pjv = jnp.where(lanes == k, pj, pjv)
        pi_buf[pl.ds(t * 16, 16)] = piv
        pj_buf[pl.ds(t * 16, 16)] = pjv
        return 0
    lax.fori_loop(0, BPW // 16, dot16, 0)
    pltpu.sync_copy(pi_buf, out_i.at[pl.ds(b0, BPW)])
    pltpu.sync_copy(pj_buf, out_j.at[pl.ds(b0, BPW)])


_score_kernel = functools.partial(
    pl.kernel,
    out_type=(
        jax.ShapeDtypeStruct((B,), jnp.float32),
        jax.ShapeDtypeStruct((B,), jnp.float32),
        jax.ShapeDtypeStruct((32, 16), jnp.float32),
    ),
    mesh=_mesh,
    scratch_types=[
        pltpu.VMEM((BPW,), jnp.int32),
        pltpu.VMEM((BPW,), jnp.int32),
        pltpu.VMEM((BPW,), jnp.int32),
        pltpu.VMEM((BPW,), jnp.int32),
        pltpu.VMEM((BPW, H), jnp.float32),
        pltpu.VMEM((BPW, H), jnp.float32),
        pltpu.VMEM((BPW, H), jnp.float32),
        pltpu.VMEM((BPW, H), jnp.float32),
        pltpu.VMEM((BPW, H), jnp.float32),
        pltpu.VMEM((BPW, H), jnp.float32),
        pltpu.VMEM((BPW, H), jnp.float32),
        pltpu.VMEM((BPW,), jnp.float32),
        pltpu.VMEM((BPW,), jnp.float32),
        pltpu.VMEM((16,), jnp.float32),
        pltpu.SemaphoreType.DMA,
    ],
    compiler_params=_params,
)(_score_body)


def kernel(user, item_i, item_j, timestamp, split_idx,
           embed_user_0, embed_item_0, graph_src, graph_dst, graph_val):
    t0 = jnp.concatenate([embed_user_0, embed_item_0], axis=0)
    t0 = jnp.concatenate(
        [t0, jnp.zeros((NP - NN, D), jnp.float32)], axis=0)
    t0s = jnp.concatenate([t0[:, :H], t0[:, H:]], axis=0)

    epad = EPAD - NEDGES
    zi = jnp.zeros((epad,), jnp.int32)
    src = jnp.concatenate([graph_src.astype(jnp.int32), zi])
    dst = jnp.concatenate([graph_dst.astype(jnp.int32), zi])
    val = jnp.concatenate([graph_val, jnp.zeros((epad,), jnp.float32)])
    vbits = jax.lax.bitcast_convert_type(val, jnp.int32)
    pk = jnp.stack([src.reshape(16 * NU, GB),
                    dst.reshape(16 * NU, GB),
                    vbits.reshape(16 * NU, GB)], axis=1)

    ot = _prop_kernel(t0s, pk)

    iu = user.astype(jnp.int32)
    ii = item_i.astype(jnp.int32) + NUSERS
    ij = item_j.astype(jnp.int32) + NUSERS
    pred_i, pred_j, reg_parts = _score_kernel(ot, iu, ii, ij)
    reg_loss = 0.5 * jnp.sum(reg_parts) / float(B)
    return pred_i, pred_j, reg_loss


# speedup vs baseline: 1.5026x; 1.5026x over previous
import functools

import jax
import jax.numpy as jnp
from jax import lax
from jax.experimental import pallas as pl
from jax.experimental.pallas import tpu as pltpu
from jax.experimental.pallas import tpu_sc as plsc

NUSERS = 25000
NITEMS = 25000
NN = NUSERS + NITEMS
NP = 50048
D = 64
H = 32
NLAYERS = 3
NEDGES = 800000
GB = 256
NSLOT = 3
NU = 198
NSI = NU // NSLOT
EPT = NU * GB
EPAD = 16 * EPT
RPT = NP // 16
ZC = 136
NZ = RPT // ZC
B = 4096
BPW = B // 32

_mesh = plsc.VectorSubcoreMesh(core_axis_name="c", subcore_axis_name="s")
_params = pltpu.CompilerParams(use_tc_tiling_on_sc=False,
                               needs_layout_passes=False)


def _prop_body(t0_hbm, pk_hbm, ot_hbm,
               acc, pb0, pb1, pb2, rows0, rows1, rows2,
               es0, es1, es2, gs0, gs1, gs2, ss0, ss1, ss2):
    pbs = (pb0, pb1, pb2)
    rws = (rows0, rows1, rows2)
    esem = (es0, es1, es2)
    gsem = (gs0, gs1, gs2)
    ssem = (ss0, ss1, ss2)
    c = lax.axis_index("c")
    w = lax.axis_index("s")
    zv = jnp.zeros((16,), jnp.float32)
    z_dma = rows0.at[pl.ds(0, ZC)]
    b_dma = rows1.at[pl.ds(0, ZC)]

    def zero_head(i, _):
        rows0[i, pl.ds(0, 16)] = zv
        rows0[i, pl.ds(16, 16)] = zv
        return 0

    lax.fori_loop(0, ZC, zero_head, 0)

    def pro(j, _):
        r0 = w * RPT + j * ZC
        pltpu.sync_copy(t0_hbm.at[pl.ds(c * NP + r0, ZC)], b_dma)
        pltpu.sync_copy(b_dma, ot_hbm.at[pl.ds(c * NP + r0, ZC)])
        pltpu.sync_copy(z_dma, acc.at[pl.ds(r0, ZC)])
        return 0
    lax.fori_loop(0, NZ, pro, 0)
    plsc.subcore_barrier()

    ubase = w * NU

    def layer(l, _):
        g_off = (2 * l + c) * NP

        def addoff(pb):
            def ao(i, _):
                sl = pl.ds(i * 16, 16)
                pb[0, sl] = pb[0, sl] + g_off
                return 0
            lax.fori_loop(0, GB // 16, ao, 0)

        def scale(pb, rows):
            def s16(t, _):
                vv = plsc.bitcast(pb[2, pl.ds(t * 16, 16)], jnp.float32)
                sa = pl.ds(0, 16)
                sb = pl.ds(16, 16)
                for k in range(16):
                    j = t * 16 + k
                    v = vv[k]
                    rows[j, sa] = rows[j, sa] * v
                    rows[j, sb] = rows[j, sb] * v
                return 0
            lax.fori_loop(0, GB // 16, s16, 0)

        pltpu.async_copy(pk_hbm.at[ubase], pb0, es0)
        pltpu.async_copy(pk_hbm.at[ubase + 1], pb1, es1)
        pltpu.make_async_copy(pk_hbm.at[ubase], pb0, es0).wait()
        addoff(pb0)
        pltpu.async_copy(ot_hbm.at[pb0.at[0]], rows0, gs0)

        def step(si, b):
            u = si * NSLOT + b
            bn = (b + 1) % NSLOT
            bp = (b + 2) % NSLOT
            pltpu.make_async_copy(ot_hbm.at[pbs[b].at[0]], rws[b],
                                  gsem[b]).wait()

            def fire_next_gather():
                pltpu.make_async_copy(pk_hbm.at[ubase], pbs[bn],
                                      esem[bn]).wait()
                addoff(pbs[bn])
                pltpu.async_copy(ot_hbm.at[pbs[bn].at[0]], rws[bn],
                                 gsem[bn])
            if b == NSLOT - 1:
                pl.when(si < NSI - 1)(fire_next_gather)
            else:
                fire_next_gather()

            scale(pbs[b], rws[b])
            pltpu.async_copy(rws[b], acc.at[pbs[b].at[1]], ssem[b],
                             add=True)

            def wait_prev_scatter():
                pltpu.make_async_copy(rws[bp], acc.at[pbs[bp].at[1]],
                                      ssem[bp]).wait()
            if b == 0:
                pl.when(si > 0)(wait_prev_scatter)
            else:
                wait_prev_scatter()

            def fire_e():
                pltpu.async_copy(pk_hbm.at[ubase + u + 2], pbs[bp],
                                 esem[bp])
            if b == 0:
                fire_e()
            else:
                pl.when(si < NSI - 1)(fire_e)

        def si_loop(si, _):
            for b in range(NSLOT):
                step(si, b)
            return 0
        lax.fori_loop(0, NSI, si_loop, 0)
        pltpu.make_async_copy(rws[NSLOT - 1],
                              acc.at[pbs[NSLOT - 1].at[1]],
                              ssem[NSLOT - 1]).wait()
        plsc.subcore_barrier()

        lax.fori_loop(0, ZC, zero_head, 0)
        o_off = (2 * (l + 1) + c) * NP

        def cz(j, _):
            r0 = w * RPT + j * ZC
            pltpu.sync_copy(acc.at[pl.ds(r0, ZC)], b_dma)
            pltpu.sync_copy(b_dma, ot_hbm.at[pl.ds(o_off + r0, ZC)])
            pltpu.sync_copy(z_dma, acc.at[pl.ds(r0, ZC)])
            return 0
        lax.fori_loop(0, NZ, cz, 0)
        plsc.subcore_barrier()
        return 0
    lax.fori_loop(0, NLAYERS, layer, 0)

    m_off = (8 + c) * NP

    def mean(j, _):
        r0 = w * RPT + j * ZC
        pltpu.sync_copy(ot_hbm.at[pl.ds(c * NP + r0, ZC)], b_dma)
        for l in range(1, 4):
            pltpu.sync_copy(ot_hbm.at[pl.ds((2 * l + c) * NP + r0, ZC)],
                            z_dma)

            def macc(r, _):
                sa = pl.ds(0, 16)
                sb = pl.ds(16, 16)
                rows1[r, sa] = rows1[r, sa] + rows0[r, sa]
                rows1[r, sb] = rows1[r, sb] + rows0[r, sb]
                return 0
            lax.fori_loop(0, ZC, macc, 0)

        def mscale(r, _):
            sa = pl.ds(0, 16)
            sb = pl.ds(16, 16)
            rows1[r, sa] = rows1[r, sa] * 0.25
            rows1[r, sb] = rows1[r, sb] * 0.25
            return 0
        lax.fori_loop(0, ZC, mscale, 0)
        pltpu.sync_copy(b_dma, ot_hbm.at[pl.ds(m_off + r0, ZC)])
        return 0
    lax.fori_loop(0, NZ, mean, 0)


_prop_kernel = functools.partial(
    pl.kernel,
    out_type=jax.ShapeDtypeStruct((10 * NP, H), jnp.float32),
    mesh=_mesh,
    scratch_types=[
        pltpu.VMEM_SHARED((NP, H), jnp.float32),
        pltpu.VMEM((3, GB), jnp.int32),
        pltpu.VMEM((3, GB), jnp.int32),
        pltpu.VMEM((3, GB), jnp.int32),
        pltpu.VMEM((GB, H), jnp.float32),
        pltpu.VMEM((GB, H), jnp.float32),
        pltpu.VMEM((GB, H), jnp.float32),
        pltpu.SemaphoreType.DMA,
        pltpu.SemaphoreType.DMA,
        pltpu.SemaphoreType.DMA,
        pltpu.SemaphoreType.DMA,
        pltpu.SemaphoreType.DMA,
        pltpu.SemaphoreType.DMA,
        pltpu.SemaphoreType.DMA,
        pltpu.SemaphoreType.DMA,
        pltpu.SemaphoreType.DMA,
    ],
    compiler_params=_params,
)(_prop_body)


def _score_body(ot_hbm, iu_hbm, ii_hbm, ij_hbm, out_i, out_j, out_r,
                b_iu, b_ii, b_ij, gidx, g,
                ulo, uhi, ilo, ihi, jlo, jhi, pi_buf, pj_buf, rbuf, sem):
    c = lax.axis_index("c")
    s = lax.axis_index("s")
    wid = s * 2 + c
    b0 = wid * BPW
    pltpu.sync_copy(iu_hbm.at[pl.ds(b0, BPW)], b_iu)
    pltpu.sync_copy(ii_hbm.at[pl.ds(b0, BPW)], b_ii)
    pltpu.sync_copy(ij_hbm.at[pl.ds(b0, BPW)], b_ij)

    def gather_to(idxbuf, off, dstbuf):
        def addoff(i, _):
            sl = pl.ds(i * 16, 16)
            gidx[sl] = idxbuf[sl] + off
            return 0
        lax.fori_loop(0, BPW // 16, addoff, 0)
        pltpu.async_copy(ot_hbm.at[gidx], dstbuf, sem).wait()

    gather_to(b_iu, 8 * NP, ulo)
    gather_to(b_iu, 9 * NP, uhi)
    gather_to(b_ii, 8 * NP, ilo)
    gather_to(b_ii, 9 * NP, ihi)
    gather_to(b_ij, 8 * NP, jlo)
    gather_to(b_ij, 9 * NP, jhi)

    racc = jnp.zeros((16,), jnp.float32)
    for idxbuf in (b_iu, b_ii, b_ij):
        for h in (0, 1):
            gather_to(idxbuf, h * NP, g)

            def sq(r, a):
                va = g[r, pl.ds(0, 16)]
                vb = g[r, pl.ds(16, 16)]
                return a + va * va + vb * vb
            racc = lax.fori_loop(0, BPW, sq, racc)
    rbuf[pl.ds(0, 16)] = racc
    pltpu.sync_copy(rbuf, out_r.at[wid])

    lanes = lax.iota(jnp.int32, 16)

    def dot16(t, _):
        s0 = pl.ds(0, 16)
        s1 = pl.ds(16, 16)
        piv = jnp.zeros((16,), jnp.float32)
        pjv = jnp.zeros((16,), jnp.float32)
        for k in range(16):
            r = t * 16 + k
            u0 = ulo[r, s0]
            u1 = ulo[r, s1]
            u2 = uhi[r, s0]
            u3 = uhi[r, s1]
            pi = jnp.sum(u0 * ilo[r, s0] + u1 * ilo[r, s1]
                         + u2 * ihi[r, s0] + u3 * ihi[r, s1])
            pj = jnp.sum(u0 * jlo[r, s0] + u1 * jlo[r, s1]
                         + u2 * jhi[r, s0] + u3 * jhi[r, s1])
            piv = jnp.where(lanes == k, pi, piv)
            pjv = jnp.where(lanes == k, pj, pjv)
        pi_buf[pl.ds(t * 16, 16)] = piv
        pj_buf[pl.ds(t * 16, 16)] = pjv
        return 0
    lax.fori_loop(0, BPW // 16, dot16, 0)
    pltpu.sync_copy(pi_buf, out_i.at[pl.ds(b0, BPW)])
    pltpu.sync_copy(pj_buf, out_j.at[pl.ds(b0, BPW)])


_score_kernel = functools.partial(
    pl.kernel,
    out_type=(
        jax.ShapeDtypeStruct((B,), jnp.float32),
        jax.ShapeDtypeStruct((B,), jnp.float32),
        jax.ShapeDtypeStruct((32, 16), jnp.float32),
    ),
    mesh=_mesh,
    scratch_types=[
        pltpu.VMEM((BPW,), jnp.int32),
        pltpu.VMEM((BPW,), jnp.int32),
        pltpu.VMEM((BPW,), jnp.int32),
        pltpu.VMEM((BPW,), jnp.int32),
        pltpu.VMEM((BPW, H), jnp.float32),
        pltpu.VMEM((BPW, H), jnp.float32),
        pltpu.VMEM((BPW, H), jnp.float32),
        pltpu.VMEM((BPW, H), jnp.float32),
        pltpu.VMEM((BPW, H), jnp.float32),
        pltpu.VMEM((BPW, H), jnp.float32),
        pltpu.VMEM((BPW, H), jnp.float32),
        pltpu.VMEM((BPW,), jnp.float32),
        pltpu.VMEM((BPW,), jnp.float32),
        pltpu.VMEM((16,), jnp.float32),
        pltpu.SemaphoreType.DMA,
    ],
    compiler_params=_params,
)(_score_body)


def kernel(user, item_i, item_j, timestamp, split_idx,
           embed_user_0, embed_item_0, graph_src, graph_dst, graph_val):
    t0 = jnp.concatenate([embed_user_0, embed_item_0], axis=0)
    t0 = jnp.concatenate(
        [t0, jnp.zeros((NP - NN, D), jnp.float32)], axis=0)
    t0s = jnp.concatenate([t0[:, :H], t0[:, H:]], axis=0)

    epad = EPAD - NEDGES
    zi = jnp.zeros((epad,), jnp.int32)
    src = jnp.concatenate([graph_src.astype(jnp.int32), zi])
    dst = jnp.concatenate([graph_dst.astype(jnp.int32), zi])
    val = jnp.concatenate([graph_val, jnp.zeros((epad,), jnp.float32)])
    vbits = jax.lax.bitcast_convert_type(val, jnp.int32)
    pk = jnp.stack([src.reshape(16 * NU, GB),
                    dst.reshape(16 * NU, GB),
                    vbits.reshape(16 * NU, GB)], axis=1)

    ot = _prop_kernel(t0s, pk)

    iu = user.astype(jnp.int32)
    ii = item_i.astype(jnp.int32) + NUSERS
    ij = item_j.astype(jnp.int32) + NUSERS
    pred_i, pred_j, reg_parts = _score_kernel(ot, iu, ii, ij)
    reg_loss = 0.5 * jnp.sum(reg_parts) / float(B)
    return pred_i, pred_j, reg_loss


# speedup vs baseline: 1.7183x; 1.1435x over previous
import functools

import jax
import jax.numpy as jnp
from jax import lax
from jax.experimental import pallas as pl
from jax.experimental.pallas import tpu as pltpu
from jax.experimental.pallas import tpu_sc as plsc

NUSERS = 25000
NITEMS = 25000
NN = NUSERS + NITEMS
NP = 50048
D = 64
H = 32
NLAYERS = 3
NEDGES = 800000
GB = 208
NSLOT = 4
NU = 244
NSI = NU // NSLOT
EPT = NU * GB
EPAD = 16 * EPT
RPT = NP // 16
ZC = 136
NZ = RPT // ZC
B = 4096
BPW = B // 32

_mesh = plsc.VectorSubcoreMesh(core_axis_name="c", subcore_axis_name="s")
_params = pltpu.CompilerParams(use_tc_tiling_on_sc=False,
                               needs_layout_passes=False)


def _prop_body(t0_hbm, pk_hbm, ot_hbm,
               acc, pb0, pb1, pb2, pb3, rows0, rows1, rows2, rows3,
               dw0, dw1,
               es0, es1, es2, es3, gs0, gs1, gs2, gs3, ss0, ss1):
    pbs = (pb0, pb1, pb2, pb3)
    rws = (rows0, rows1, rows2, rows3)
    dws = (dw0, dw1)
    esem = (es0, es1, es2, es3)
    gsem = (gs0, gs1, gs2, gs3)
    ssem = (ss0, ss1)
    c = lax.axis_index("c")
    w = lax.axis_index("s")
    zv = jnp.zeros((16,), jnp.float32)
    z_dma = rows0.at[pl.ds(0, ZC)]
    b_dma = rows1.at[pl.ds(0, ZC)]

    def zero_head(i, _):
        rows0[i, pl.ds(0, 16)] = zv
        rows0[i, pl.ds(16, 16)] = zv
        return 0

    lax.fori_loop(0, ZC, zero_head, 0)

    def pro(j, _):
        r0 = w * RPT + j * ZC
        pltpu.sync_copy(t0_hbm.at[pl.ds(c * NP + r0, ZC)], b_dma)
        pltpu.sync_copy(b_dma, ot_hbm.at[pl.ds(c * NP + r0, ZC)])
        pltpu.sync_copy(z_dma, acc.at[pl.ds(r0, ZC)])
        return 0
    lax.fori_loop(0, NZ, pro, 0)
    plsc.subcore_barrier()

    ubase = w * NU

    def layer(l, _):
        g_off = (2 * l + c) * NP

        def addoff(pb):
            def ao(i, _):
                sl = pl.ds(i * 16, 16)
                pb[0, sl] = pb[0, sl] + g_off
                return 0
            lax.fori_loop(0, GB // 16, ao, 0)

        def scale(pb, rows):
            def s16(t, _):
                vv = plsc.bitcast(pb[2, pl.ds(t * 16, 16)], jnp.float32)
                sa = pl.ds(0, 16)
                sb = pl.ds(16, 16)
                for k in range(16):
                    j = t * 16 + k
                    v = vv[k]
                    rows[j, sa] = rows[j, sa] * v
                    rows[j, sb] = rows[j, sb] * v
                return 0
            lax.fori_loop(0, GB // 16, s16, 0)

        def cpdst(pb, dw):
            def cd(i, _):
                sl = pl.ds(i * 16, 16)
                dw[sl] = pb[1, sl]
                return 0
            lax.fori_loop(0, GB // 16, cd, 0)

        for q in range(NSLOT):
            pltpu.async_copy(pk_hbm.at[ubase + q], pbs[q], esem[q])
        for q in range(2):
            pltpu.make_async_copy(pk_hbm.at[ubase], pbs[q],
                                  esem[q]).wait()
            addoff(pbs[q])
            pltpu.async_copy(ot_hbm.at[pbs[q].at[0]], rws[q], gsem[q])

        def step(si, b):
            u = si * NSLOT + b
            b2 = (b + 2) % NSLOT
            w2 = b % 2
            pltpu.make_async_copy(ot_hbm.at[pbs[b].at[0]], rws[b],
                                  gsem[b]).wait()

            def drain_w():
                pltpu.make_async_copy(rws[b2], acc.at[dws[w2]],
                                      ssem[w2]).wait()
            if b < 2:
                pl.when(si > 0)(drain_w)
            else:
                drain_w()

            def prep_next():
                pltpu.make_async_copy(pk_hbm.at[ubase], pbs[b2],
                                      esem[b2]).wait()
                addoff(pbs[b2])
                pltpu.async_copy(ot_hbm.at[pbs[b2].at[0]], rws[b2],
                                 gsem[b2])
            if b < 2:
                prep_next()
            else:
                pl.when(si < NSI - 1)(prep_next)

            scale(pbs[b], rws[b])
            cpdst(pbs[b], dws[w2])
            pltpu.async_copy(rws[b], acc.at[dws[w2]], ssem[w2],
                             add=True)

            def fire_e():
                pltpu.async_copy(pk_hbm.at[ubase + u + 4], pbs[b],
                                 esem[b])
            pl.when(si < NSI - 1)(fire_e)

        def si_loop(si, _):
            for b in range(NSLOT):
                step(si, b)
            return 0
        lax.fori_loop(0, NSI, si_loop, 0)
        pltpu.make_async_copy(rws[2], acc.at[dws[0]], ssem[0]).wait()
        pltpu.make_async_copy(rws[3], acc.at[dws[1]], ssem[1]).wait()
        plsc.subcore_barrier()

        lax.fori_loop(0, ZC, zero_head, 0)
        o_off = (2 * (l + 1) + c) * NP

        def cz(j, _):
            r0 = w * RPT + j * ZC
            pltpu.sync_copy(acc.at[pl.ds(r0, ZC)], b_dma)
            pltpu.sync_copy(b_dma, ot_hbm.at[pl.ds(o_off + r0, ZC)])
            pltpu.sync_copy(z_dma, acc.at[pl.ds(r0, ZC)])
            return 0
        lax.fori_loop(0, NZ, cz, 0)
        plsc.subcore_barrier()
        return 0
    lax.fori_loop(0, NLAYERS, layer, 0)

    m_off = (8 + c) * NP

    def mean(j, _):
        r0 = w * RPT + j * ZC
        pltpu.sync_copy(ot_hbm.at[pl.ds(c * NP + r0, ZC)], b_dma)
        for l in range(1, 4):
            pltpu.sync_copy(ot_hbm.at[pl.ds((2 * l + c) * NP + r0, ZC)],
                            z_dma)

            def macc(r, _):
                sa = pl.ds(0, 16)
                sb = pl.ds(16, 16)
                rows1[r, sa] = rows1[r, sa] + rows0[r, sa]
                rows1[r, sb] = rows1[r, sb] + rows0[r, sb]
                return 0
            lax.fori_loop(0, ZC, macc, 0)

        def mscale(r, _):
            sa = pl.ds(0, 16)
            sb = pl.ds(16, 16)
            rows1[r, sa] = rows1[r, sa] * 0.25
            rows1[r, sb] = rows1[r, sb] * 0.25
            return 0
        lax.fori_loop(0, ZC, mscale, 0)
        pltpu.sync_copy(b_dma, ot_hbm.at[pl.ds(m_off + r0, ZC)])
        return 0
    lax.fori_loop(0, NZ, mean, 0)


_prop_kernel = functools.partial(
    pl.kernel,
    out_type=jax.ShapeDtypeStruct((10 * NP, H), jnp.float32),
    mesh=_mesh,
    scratch_types=[
        pltpu.VMEM_SHARED((NP, H), jnp.float32),
        pltpu.VMEM((3, GB), jnp.int32),
        pltpu.VMEM((3, GB), jnp.int32),
        pltpu.VMEM((3, GB), jnp.int32),
        pltpu.VMEM((3, GB), jnp.int32),
        pltpu.VMEM((GB, H), jnp.float32),
        pltpu.VMEM((GB, H), jnp.float32),
        pltpu.VMEM((GB, H), jnp.float32),
        pltpu.VMEM((GB, H), jnp.float32),
        pltpu.VMEM((GB,), jnp.int32),
        pltpu.VMEM((GB,), jnp.int32),
        pltpu.SemaphoreType.DMA,
        pltpu.SemaphoreType.DMA,
        pltpu.SemaphoreType.DMA,
        pltpu.SemaphoreType.DMA,
        pltpu.SemaphoreType.DMA,
        pltpu.SemaphoreType.DMA,
        pltpu.SemaphoreType.DMA,
        pltpu.SemaphoreType.DMA,
        pltpu.SemaphoreType.DMA,
        pltpu.SemaphoreType.DMA,
    ],
    compiler_params=_params,
)(_prop_body)


def _score_body(ot_hbm, iu_hbm, ii_hbm, ij_hbm, out_i, out_j, out_r,
                b_iu, b_ii, b_ij, gidx, g,
                ulo, uhi, ilo, ihi, jlo, jhi, pi_buf, pj_buf, rbuf, sem):
    c = lax.axis_index("c")
    s = lax.axis_index("s")
    wid = s * 2 + c
    b0 = wid * BPW
    pltpu.sync_copy(iu_hbm.at[pl.ds(b0, BPW)], b_iu)
    pltpu.sync_copy(ii_hbm.at[pl.ds(b0, BPW)], b_ii)
    pltpu.sync_copy(ij_hbm.at[pl.ds(b0, BPW)], b_ij)

    def gather_to(idxbuf, off, dstbuf):
        def addoff(i, _):
            sl = pl.ds(i * 16, 16)
            gidx[sl] = idxbuf[sl] + off
            return 0
        lax.fori_loop(0, BPW // 16, addoff, 0)
        pltpu.async_copy(ot_hbm.at[gidx], dstbuf, sem).wait()

    gather_to(b_iu, 8 * NP, ulo)
    gather_to(b_iu, 9 * NP, uhi)
    gather_to(b_ii, 8 * NP, ilo)
    gather_to(b_ii, 9 * NP, ihi)
    gather_to(b_ij, 8 * NP, jlo)
    gather_to(b_ij, 9 * NP, jhi)

    racc = jnp.zeros((16,), jnp.float32)
    for idxbuf in (b_iu, b_ii, b_ij):
        for h in (0, 1):
            gather_to(idxbuf, h * NP, g)

            def sq(r, a):
                va = g[r, pl.ds(0, 16)]
                vb = g[r, pl.ds(16, 16)]
                return a + va * va + vb * vb
            racc = lax.fori_loop(0, BPW, sq, racc)
    rbuf[pl.ds(0, 16)] = racc
    pltpu.sync_copy(rbuf, out_r.at[wid])

    lanes = lax.iota(jnp.int32, 16)

    def dot16(t, _):
        s0 = pl.ds(0, 16)
        s1 = pl.ds(16, 16)
        piv = jnp.zeros((16,), jnp.float32)
        pjv = jnp.zeros((16,), jnp.float32)
        for k in range(16):
            r = t * 16 + k
            u0 = ulo[r, s0]
            u1 = ulo[r, s1]
            u2 = uhi[r, s0]
            u3 = uhi[r, s1]
            pi = jnp.sum(u0 * ilo[r, s0] + u1 * ilo[r, s1]
                         + u2 * ihi[r, s0] + u3 * ihi[r, s1])
            pj = jnp.sum(u0 * jlo[r, s0] + u1 * jlo[r, s1]
                         + u2 * jhi[r, s0] + u3 * jhi[r, s1])
            piv = jnp.where(lanes == k, pi, piv)
            pjv = jnp.where(lanes == k, pj, pjv)
        pi_buf[pl.ds(t * 16, 16)] = piv
        pj_buf[pl.ds(t * 16, 16)] = pjv
        return 0
    lax.fori_loop(0, BPW // 16, dot16, 0)
    pltpu.sync_copy(pi_buf, out_i.at[pl.ds(b0, BPW)])
    pltpu.sync_copy(pj_buf, out_j.at[pl.ds(b0, BPW)])


_score_kernel = functools.partial(
    pl.kernel,
    out_type=(
        jax.ShapeDtypeStruct((B,), jnp.float32),
        jax.ShapeDtypeStruct((B,), jnp.float32),
        jax.ShapeDtypeStruct((32, 16), jnp.float32),
    ),
    mesh=_mesh,
    scratch_types=[
        pltpu.VMEM((BPW,), jnp.int32),
        pltpu.VMEM((BPW,), jnp.int32),
        pltpu.VMEM((BPW,), jnp.int32),
        pltpu.VMEM((BPW,), jnp.int32),
        pltpu.VMEM((BPW, H), jnp.float32),
        pltpu.VMEM((BPW, H), jnp.float32),
        pltpu.VMEM((BPW, H), jnp.float32),
        pltpu.VMEM((BPW, H), jnp.float32),
        pltpu.VMEM((BPW, H), jnp.float32),
        pltpu.VMEM((BPW, H), jnp.float32),
        pltpu.VMEM((BPW, H), jnp.float32),
        pltpu.VMEM((BPW,), jnp.float32),
        pltpu.VMEM((BPW,), jnp.float32),
        pltpu.VMEM((16,), jnp.float32),
        pltpu.SemaphoreType.DMA,
    ],
    compiler_params=_params,
)(_score_body)


def kernel(user, item_i, item_j, timestamp, split_idx,
           embed_user_0, embed_item_0, graph_src, graph_dst, graph_val):
    t0 = jnp.concatenate([embed_user_0, embed_item_0], axis=0)
    t0 = jnp.concatenate(
        [t0, jnp.zeros((NP - NN, D), jnp.float32)], axis=0)
    t0s = jnp.concatenate([t0[:, :H], t0[:, H:]], axis=0)

    epad = EPAD - NEDGES
    zi = jnp.zeros((epad,), jnp.int32)
    src = jnp.concatenate([graph_src.astype(jnp.int32), zi])
    dst = jnp.concatenate([graph_dst.astype(jnp.int32), zi])
    val = jnp.concatenate([graph_val, jnp.zeros((epad,), jnp.float32)])
    vbits = jax.lax.bitcast_convert_type(val, jnp.int32)
    pk = jnp.stack([src.reshape(16 * NU, GB),
                    dst.reshape(16 * NU, GB),
                    vbits.reshape(16 * NU, GB)], axis=1)

    ot = _prop_kernel(t0s, pk)

    iu = user.astype(jnp.int32)
    ii = item_i.astype(jnp.int32) + NUSERS
    ij = item_j.astype(jnp.int32) + NUSERS
    pred_i, pred_j, reg_parts = _score_kernel(ot, iu, ii, ij)
    reg_loss = 0.5 * jnp.sum(reg_parts) / float(B)
    return pred_i, pred_j, reg_loss


# speedup vs baseline: 1.7731x; 1.0319x over previous
import functools

import jax
import jax.numpy as jnp
from jax import lax
from jax.experimental import pallas as pl
from jax.experimental.pallas import tpu as pltpu
from jax.experimental.pallas import tpu_sc as plsc

NUSERS = 25000
NITEMS = 25000
NN = NUSERS + NITEMS
NP = 50048
D = 64
H = 32
NLAYERS = 3
NEDGES = 800000
GB = 208
NSLOT = 4
NU = 244
NSI = NU // NSLOT
EPT = NU * GB
EPAD = 16 * EPT
RPT = NP // 16
ZC = 184
NZ = RPT // ZC
B = 4096
BPW = B // 32

_mesh = plsc.VectorSubcoreMesh(core_axis_name="c", subcore_axis_name="s")
_params = pltpu.CompilerParams(use_tc_tiling_on_sc=False,
                               needs_layout_passes=False)


def _prop_body(t0_hbm, pk_hbm, ot_hbm,
               acc, pb0, pb1, pb2, pb3, rows0, rows1, rows2, rows3,
               dw0, dw1,
               es0, es1, es2, es3, gs0, gs1, gs2, gs3, ss0, ss1):
    pbs = (pb0, pb1, pb2, pb3)
    rws = (rows0, rows1, rows2, rows3)
    dws = (dw0, dw1)
    esem = (es0, es1, es2, es3)
    gsem = (gs0, gs1, gs2, gs3)
    ssem = (ss0, ss1)
    c = lax.axis_index("c")
    w = lax.axis_index("s")
    zv = jnp.zeros((16,), jnp.float32)
    z_dma = rows0.at[pl.ds(0, ZC)]
    b_dma = rows1.at[pl.ds(0, ZC)]

    def zero_head(i, _):
        rows0[i, pl.ds(0, 16)] = zv
        rows0[i, pl.ds(16, 16)] = zv
        return 0

    lax.fori_loop(0, ZC, zero_head, 0)

    def pro(j, _):
        r0 = w * RPT + j * ZC
        pltpu.sync_copy(t0_hbm.at[pl.ds(c * NP + r0, ZC)], b_dma)
        pltpu.sync_copy(b_dma, ot_hbm.at[pl.ds(c * NP + r0, ZC)])
        pltpu.sync_copy(z_dma, acc.at[pl.ds(r0, ZC)])
        return 0
    lax.fori_loop(0, NZ, pro, 0)
    plsc.subcore_barrier()

    ubase = w * NU

    def layer(l, _):
        g_off = (2 * l + c) * NP

        def addoff(pb):
            def ao(i, _):
                sl = pl.ds(i * 16, 16)
                pb[0, sl] = pb[0, sl] + g_off
                return 0
            lax.fori_loop(0, GB // 16, ao, 0)

        def scale(pb, rows):
            def s16(t, _):
                vv = plsc.bitcast(pb[2, pl.ds(t * 16, 16)], jnp.float32)
                sa = pl.ds(0, 16)
                sb = pl.ds(16, 16)
                for k in range(16):
                    j = t * 16 + k
                    v = vv[k]
                    rows[j, sa] = rows[j, sa] * v
                    rows[j, sb] = rows[j, sb] * v
                return 0
            lax.fori_loop(0, GB // 16, s16, 0)

        def cpdst(pb, dw):
            def cd(i, _):
                sl = pl.ds(i * 16, 16)
                dw[sl] = pb[1, sl]
                return 0
            lax.fori_loop(0, GB // 16, cd, 0)

        for q in range(NSLOT):
            pltpu.async_copy(pk_hbm.at[ubase + q], pbs[q], esem[q])
        for q in range(2):
            pltpu.make_async_copy(pk_hbm.at[ubase], pbs[q],
                                  esem[q]).wait()
            addoff(pbs[q])
            pltpu.async_copy(ot_hbm.at[pbs[q].at[0]], rws[q], gsem[q])

        def step(si, b):
            u = si * NSLOT + b
            b2 = (b + 2) % NSLOT
            w2 = b % 2
            pltpu.make_async_copy(ot_hbm.at[pbs[b].at[0]], rws[b],
                                  gsem[b]).wait()

            def drain_w():
                pltpu.make_async_copy(rws[b2], acc.at[dws[w2]],
                                      ssem[w2]).wait()
            if b < 2:
                pl.when(si > 0)(drain_w)
            else:
                drain_w()

            def prep_next():
                pltpu.make_async_copy(pk_hbm.at[ubase], pbs[b2],
                                      esem[b2]).wait()
                addoff(pbs[b2])
                pltpu.async_copy(ot_hbm.at[pbs[b2].at[0]], rws[b2],
                                 gsem[b2])
            if b < 2:
                prep_next()
            else:
                pl.when(si < NSI - 1)(prep_next)

            scale(pbs[b], rws[b])
            cpdst(pbs[b], dws[w2])
            pltpu.async_copy(rws[b], acc.at[dws[w2]], ssem[w2],
                             add=True)

            def fire_e():
                pltpu.async_copy(pk_hbm.at[ubase + u + 4], pbs[b],
                                 esem[b])
            pl.when(si < NSI - 1)(fire_e)

        def si_loop(si, _):
            for b in range(NSLOT):
                step(si, b)
            return 0
        lax.fori_loop(0, NSI, si_loop, 0)
        pltpu.make_async_copy(rws[2], acc.at[dws[0]], ssem[0]).wait()
        pltpu.make_async_copy(rws[3], acc.at[dws[1]], ssem[1]).wait()
        plsc.subcore_barrier()

        lax.fori_loop(0, ZC, zero_head, 0)
        o_off = (2 * (l + 1) + c) * NP
        m_off = (8 + c) * NP
        m_dma = rows2.at[pl.ds(0, ZC)]

        def cz(j, _):
            r0 = w * RPT + j * ZC
            pltpu.sync_copy(acc.at[pl.ds(r0, ZC)], b_dma)
            pltpu.sync_copy(b_dma, ot_hbm.at[pl.ds(o_off + r0, ZC)])
            pltpu.sync_copy(z_dma, acc.at[pl.ds(r0, ZC)])

            @pl.when(l == NLAYERS - 1)
            def _mean():
                for ll in range(NLAYERS):
                    pltpu.sync_copy(
                        ot_hbm.at[pl.ds((2 * ll + c) * NP + r0, ZC)],
                        m_dma)

                    def macc(r, _):
                        sa = pl.ds(0, 16)
                        sb = pl.ds(16, 16)
                        rows1[r, sa] = rows1[r, sa] + rows2[r, sa]
                        rows1[r, sb] = rows1[r, sb] + rows2[r, sb]
                        return 0
                    lax.fori_loop(0, ZC, macc, 0)

                def mscale(r, _):
                    sa = pl.ds(0, 16)
                    sb = pl.ds(16, 16)
                    rows1[r, sa] = rows1[r, sa] * 0.25
                    rows1[r, sb] = rows1[r, sb] * 0.25
                    return 0
                lax.fori_loop(0, ZC, mscale, 0)
                pltpu.sync_copy(b_dma, ot_hbm.at[pl.ds(m_off + r0, ZC)])
            return 0
        lax.fori_loop(0, NZ, cz, 0)
        plsc.subcore_barrier()
        return 0
    lax.fori_loop(0, NLAYERS, layer, 0)


_prop_kernel = functools.partial(
    pl.kernel,
    out_type=jax.ShapeDtypeStruct((10 * NP, H), jnp.float32),
    mesh=_mesh,
    scratch_types=[
        pltpu.VMEM_SHARED((NP, H), jnp.float32),
        pltpu.VMEM((3, GB), jnp.int32),
        pltpu.VMEM((3, GB), jnp.int32),
        pltpu.VMEM((3, GB), jnp.int32),
        pltpu.VMEM((3, GB), jnp.int32),
        pltpu.VMEM((GB, H), jnp.float32),
        pltpu.VMEM((GB, H), jnp.float32),
        pltpu.VMEM((GB, H), jnp.float32),
        pltpu.VMEM((GB, H), jnp.float32),
        pltpu.VMEM((GB,), jnp.int32),
        pltpu.VMEM((GB,), jnp.int32),
        pltpu.SemaphoreType.DMA,
        pltpu.SemaphoreType.DMA,
        pltpu.SemaphoreType.DMA,
        pltpu.SemaphoreType.DMA,
        pltpu.SemaphoreType.DMA,
        pltpu.SemaphoreType.DMA,
        pltpu.SemaphoreType.DMA,
        pltpu.SemaphoreType.DMA,
        pltpu.SemaphoreType.DMA,
        pltpu.SemaphoreType.DMA,
    ],
    compiler_params=_params,
)(_prop_body)


def _score_body(ot_hbm, iu_hbm, ii_hbm, ij_hbm, out_i, out_j, out_r,
                b_iu, b_ii, b_ij, gidx, g,
                ulo, uhi, ilo, ihi, jlo, jhi, pi_buf, pj_buf, rbuf, sem):
    c = lax.axis_index("c")
    s = lax.axis_index("s")
    wid = s * 2 + c
    b0 = wid * BPW
    pltpu.sync_copy(iu_hbm.at[pl.ds(b0, BPW)], b_iu)
    pltpu.sync_copy(ii_hbm.at[pl.ds(b0, BPW)], b_ii)
    pltpu.sync_copy(ij_hbm.at[pl.ds(b0, BPW)], b_ij)

    def gather_to(idxbuf, off, dstbuf):
        def addoff(i, _):
            sl = pl.ds(i * 16, 16)
            gidx[sl] = idxbuf[sl] + off
            return 0
        lax.fori_loop(0, BPW // 16, addoff, 0)
        pltpu.async_copy(ot_hbm.at[gidx], dstbuf, sem).wait()

    gather_to(b_iu, 8 * NP, ulo)
    gather_to(b_iu, 9 * NP, uhi)
    gather_to(b_ii, 8 * NP, ilo)
    gather_to(b_ii, 9 * NP, ihi)
    gather_to(b_ij, 8 * NP, jlo)
    gather_to(b_ij, 9 * NP, jhi)

    racc = jnp.zeros((16,), jnp.float32)
    for idxbuf in (b_iu, b_ii, b_ij):
        for h in (0, 1):
            gather_to(idxbuf, h * NP, g)

            def sq(r, a):
                va = g[r, pl.ds(0, 16)]
                vb = g[r, pl.ds(16, 16)]
                return a + va * va + vb * vb
            racc = lax.fori_loop(0, BPW, sq, racc)
    rbuf[pl.ds(0, 16)] = racc
    pltpu.sync_copy(rbuf, out_r.at[wid])

    lanes = lax.iota(jnp.int32, 16)

    def dot16(t, _):
        s0 = pl.ds(0, 16)
        s1 = pl.ds(16, 16)
        piv = jnp.zeros((16,), jnp.float32)
        pjv = jnp.zeros((16,), jnp.float32)
        for k in range(16):
            r = t * 16 + k
            u0 = ulo[r, s0]
            u1 = ulo[r, s1]
            u2 = uhi[r, s0]
            u3 = uhi[r, s1]
            pi = jnp.sum(u0 * ilo[r, s0] + u1 * ilo[r, s1]
                         + u2 * ihi[r, s0] + u3 * ihi[r, s1])
            pj = jnp.sum(u0 * jlo[r, s0] + u1 * jlo[r, s1]
                         + u2 * jhi[r, s0] + u3 * jhi[r, s1])
            piv = jnp.where(lanes == k, pi, piv)
            pjv = jnp.where(lanes == k, pj, pjv)
        pi_buf[pl.ds(t * 16, 16)] = piv
        pj_buf[pl.ds(t * 16, 16)] = pjv
        return 0
    lax.fori_loop(0, BPW // 16, dot16, 0)
    pltpu.sync_copy(pi_buf, out_i.at[pl.ds(b0, BPW)])
    pltpu.sync_copy(pj_buf, out_j.at[pl.ds(b0, BPW)])


_score_kernel = functools.partial(
    pl.kernel,
    out_type=(
        jax.ShapeDtypeStruct((B,), jnp.float32),
        jax.ShapeDtypeStruct((B,), jnp.float32),
        jax.ShapeDtypeStruct((32, 16), jnp.float32),
    ),
    mesh=_mesh,
    scratch_types=[
        pltpu.VMEM((BPW,), jnp.int32),
        pltpu.VMEM((BPW,), jnp.int32),
        pltpu.VMEM((BPW,), jnp.int32),
        pltpu.VMEM((BPW,), jnp.int32),
        pltpu.VMEM((BPW, H), jnp.float32),
        pltpu.VMEM((BPW, H), jnp.float32),
        pltpu.VMEM((BPW, H), jnp.float32),
        pltpu.VMEM((BPW, H), jnp.float32),
        pltpu.VMEM((BPW, H), jnp.float32),
        pltpu.VMEM((BPW, H), jnp.float32),
        pltpu.VMEM((BPW, H), jnp.float32),
        pltpu.VMEM((BPW,), jnp.float32),
        pltpu.VMEM((BPW,), jnp.float32),
        pltpu.VMEM((16,), jnp.float32),
        pltpu.SemaphoreType.DMA,
    ],
    compiler_params=_params,
)(_score_body)


def kernel(user, item_i, item_j, timestamp, split_idx,
           embed_user_0, embed_item_0, graph_src, graph_dst, graph_val):
    t0 = jnp.concatenate([embed_user_0, embed_item_0], axis=0)
    t0 = jnp.concatenate(
        [t0, jnp.zeros((NP - NN, D), jnp.float32)], axis=0)
    t0s = jnp.concatenate([t0[:, :H], t0[:, H:]], axis=0)

    epad = EPAD - NEDGES
    zi = jnp.zeros((epad,), jnp.int32)
    src = jnp.concatenate([graph_src.astype(jnp.int32), zi])
    dst = jnp.concatenate([graph_dst.astype(jnp.int32), zi])
    val = jnp.concatenate([graph_val, jnp.zeros((epad,), jnp.float32)])
    vbits = jax.lax.bitcast_convert_type(val, jnp.int32)
    pk = jnp.stack([src.reshape(16 * NU, GB),
                    dst.reshape(16 * NU, GB),
                    vbits.reshape(16 * NU, GB)], axis=1)

    ot = _prop_kernel(t0s, pk)

    iu = user.astype(jnp.int32)
    ii = item_i.astype(jnp.int32) + NUSERS
    ij = item_j.astype(jnp.int32) + NUSERS
    pred_i, pred_j, reg_parts = _score_kernel(ot, iu, ii, ij)
    reg_loss = 0.5 * jnp.sum(reg_parts) / float(B)
    return pred_i, pred_j, reg_loss


# speedup vs baseline: 2.4271x; 1.3688x over previous
import functools

import jax
import jax.numpy as jnp
from jax import lax
from jax.experimental import pallas as pl
from jax.experimental.pallas import tpu as pltpu
from jax.experimental.pallas import tpu_sc as plsc

NUSERS = 25000
NITEMS = 25000
NN = NUSERS + NITEMS
NP = 50048
D = 64
H = 32
NLAYERS = 3
NEDGES = 800000
GB = 144
NSLOT = 6
NU = 348
NSI = NU // NSLOT
EPT = NU * GB
EPAD = 16 * EPT
RPT = NP // 16
ZC = 136
NZ = RPT // ZC
B = 4096
BPW = B // 32

_mesh = plsc.VectorSubcoreMesh(core_axis_name="c", subcore_axis_name="s")
_params = pltpu.CompilerParams(use_tc_tiling_on_sc=False,
                               needs_layout_passes=False)


def _prop_body(t0_hbm, pk_hbm, ot_hbm,
               acc, pb0, pb1, pb2, pb3, pb4, pb5,
               rows0, rows1, rows2, rows3, rows4, rows5,
               dw0, dw1, dw2,
               es0, es1, es2, es3, es4, es5,
               gs0, gs1, gs2, gs3, gs4, gs5, ss0, ss1, ss2):
    pbs = (pb0, pb1, pb2, pb3, pb4, pb5)
    rws = (rows0, rows1, rows2, rows3, rows4, rows5)
    dws = (dw0, dw1, dw2)
    esem = (es0, es1, es2, es3, es4, es5)
    gsem = (gs0, gs1, gs2, gs3, gs4, gs5)
    ssem = (ss0, ss1, ss2)
    c = lax.axis_index("c")
    w = lax.axis_index("s")
    zv = jnp.zeros((16,), jnp.float32)
    z_dma = rows0.at[pl.ds(0, ZC)]
    b_dma = rows1.at[pl.ds(0, ZC)]

    def zero_head(i, _):
        rows0[i, pl.ds(0, 16)] = zv
        rows0[i, pl.ds(16, 16)] = zv
        return 0

    lax.fori_loop(0, ZC, zero_head, 0)

    def pro(j, _):
        r0 = w * RPT + j * ZC
        pltpu.sync_copy(t0_hbm.at[pl.ds(c * NP + r0, ZC)], b_dma)
        pltpu.sync_copy(b_dma, ot_hbm.at[pl.ds(c * NP + r0, ZC)])
        pltpu.sync_copy(z_dma, acc.at[pl.ds(r0, ZC)])
        return 0
    lax.fori_loop(0, NZ, pro, 0)
    plsc.subcore_barrier()

    ubase = w * NU

    def layer(l, _):
        g_off = (2 * l + c) * NP

        def addoff(pb):
            def ao(i, _):
                sl = pl.ds(i * 16, 16)
                pb[0, sl] = pb[0, sl] + g_off
                return 0
            lax.fori_loop(0, GB // 16, ao, 0)

        def scale(pb, rows):
            def s16(t, _):
                vv = plsc.bitcast(pb[2, pl.ds(t * 16, 16)], jnp.float32)
                sa = pl.ds(0, 16)
                sb = pl.ds(16, 16)
                for k in range(16):
                    j = t * 16 + k
                    v = vv[k]
                    rows[j, sa] = rows[j, sa] * v
                    rows[j, sb] = rows[j, sb] * v
                return 0
            lax.fori_loop(0, GB // 16, s16, 0)

        def cpdst(pb, dw):
            def cd(i, _):
                sl = pl.ds(i * 16, 16)
                dw[sl] = pb[1, sl]
                return 0
            lax.fori_loop(0, GB // 16, cd, 0)

        for q in range(NSLOT):
            pltpu.async_copy(pk_hbm.at[ubase + q], pbs[q], esem[q])
        for q in range(3):
            pltpu.make_async_copy(pk_hbm.at[ubase], pbs[q],
                                  esem[q]).wait()
            addoff(pbs[q])
            pltpu.async_copy(ot_hbm.at[pbs[q].at[0]], rws[q], gsem[q])

        def step(si, b):
            u = si * NSLOT + b
            b3 = (b + 3) % NSLOT
            w3 = b % 3
            pltpu.make_async_copy(ot_hbm.at[pbs[b].at[0]], rws[b],
                                  gsem[b]).wait()

            def drain_w():
                pltpu.make_async_copy(rws[b3], acc.at[dws[w3]],
                                      ssem[w3]).wait()
            if b < 3:
                pl.when(si > 0)(drain_w)
            else:
                drain_w()

            def prep_next():
                pltpu.make_async_copy(pk_hbm.at[ubase], pbs[b3],
                                      esem[b3]).wait()
                addoff(pbs[b3])
                pltpu.async_copy(ot_hbm.at[pbs[b3].at[0]], rws[b3],
                                 gsem[b3])
            if b < 3:
                prep_next()
            else:
                pl.when(si < NSI - 1)(prep_next)

            scale(pbs[b], rws[b])
            cpdst(pbs[b], dws[w3])
            pltpu.async_copy(rws[b], acc.at[dws[w3]], ssem[w3],
                             add=True)

            def fire_e():
                pltpu.async_copy(pk_hbm.at[ubase + u + NSLOT], pbs[b],
                                 esem[b])
            pl.when(si < NSI - 1)(fire_e)

        def si_loop(si, _):
            for b in range(NSLOT):
                step(si, b)
            return 0
        lax.fori_loop(0, NSI, si_loop, 0)
        pltpu.make_async_copy(rws[3], acc.at[dws[0]], ssem[0]).wait()
        pltpu.make_async_copy(rws[4], acc.at[dws[1]], ssem[1]).wait()
        pltpu.make_async_copy(rws[5], acc.at[dws[2]], ssem[2]).wait()
        plsc.subcore_barrier()

        lax.fori_loop(0, ZC, zero_head, 0)
        o_off = (2 * (l + 1) + c) * NP
        m_off = (8 + c) * NP
        m_dma = rows2.at[pl.ds(0, ZC)]

        def cz(j, _):
            r0 = w * RPT + j * ZC
            pltpu.sync_copy(acc.at[pl.ds(r0, ZC)], b_dma)
            pltpu.sync_copy(b_dma, ot_hbm.at[pl.ds(o_off + r0, ZC)])
            pltpu.sync_copy(z_dma, acc.at[pl.ds(r0, ZC)])

            @pl.when(l == NLAYERS - 1)
            def _mean():
                for ll in range(NLAYERS):
                    pltpu.sync_copy(
                        ot_hbm.at[pl.ds((2 * ll + c) * NP + r0, ZC)],
                        m_dma)

                    def macc(r, _):
                        sa = pl.ds(0, 16)
                        sb = pl.ds(16, 16)
                        rows1[r, sa] = rows1[r, sa] + rows2[r, sa]
                        rows1[r, sb] = rows1[r, sb] + rows2[r, sb]
                        return 0
                    lax.fori_loop(0, ZC, macc, 0)

                def mscale(r, _):
                    sa = pl.ds(0, 16)
                    sb = pl.ds(16, 16)
                    rows1[r, sa] = rows1[r, sa] * 0.25
                    rows1[r, sb] = rows1[r, sb] * 0.25
                    return 0
                lax.fori_loop(0, ZC, mscale, 0)
                pltpu.sync_copy(b_dma, ot_hbm.at[pl.ds(m_off + r0, ZC)])
            return 0
        lax.fori_loop(0, NZ, cz, 0)
        plsc.subcore_barrier()
        return 0
    lax.fori_loop(0, NLAYERS, layer, 0)


_prop_kernel = functools.partial(
    pl.kernel,
    out_type=jax.ShapeDtypeStruct((10 * NP, H), jnp.float32),
    mesh=_mesh,
    scratch_types=[
        pltpu.VMEM_SHARED((NP, H), jnp.float32),
        pltpu.VMEM((3, GB), jnp.int32),
        pltpu.VMEM((3, GB), jnp.int32),
        pltpu.VMEM((3, GB), jnp.int32),
        pltpu.VMEM((3, GB), jnp.int32),
        pltpu.VMEM((3, GB), jnp.int32),
        pltpu.VMEM((3, GB), jnp.int32),
        pltpu.VMEM((GB, H), jnp.float32),
        pltpu.VMEM((GB, H), jnp.float32),
        pltpu.VMEM((GB, H), jnp.float32),
        pltpu.VMEM((GB, H), jnp.float32),
        pltpu.VMEM((GB, H), jnp.float32),
        pltpu.VMEM((GB, H), jnp.float32),
        pltpu.VMEM((GB,), jnp.int32),
        pltpu.VMEM((GB,), jnp.int32),
        pltpu.VMEM((GB,), jnp.int32),
        pltpu.SemaphoreType.DMA,
        pltpu.SemaphoreType.DMA,
        pltpu.SemaphoreType.DMA,
        pltpu.SemaphoreType.DMA,
        pltpu.SemaphoreType.DMA,
        pltpu.SemaphoreType.DMA,
        pltpu.SemaphoreType.DMA,
        pltpu.SemaphoreType.DMA,
        pltpu.SemaphoreType.DMA,
        pltpu.SemaphoreType.DMA,
        pltpu.SemaphoreType.DMA,
        pltpu.SemaphoreType.DMA,
        pltpu.SemaphoreType.DMA,
        pltpu.SemaphoreType.DMA,
        pltpu.SemaphoreType.DMA,
    ],
    compiler_params=_params,
)(_prop_body)


def _score_body(ot_hbm, iu_hbm, ii_hbm, ij_hbm, out_i, out_j, out_r,
                b_iu, b_ii, b_ij, gidx, g,
                ulo, uhi, ilo, ihi, jlo, jhi, pi_buf, pj_buf, rbuf, sem):
    c = lax.axis_index("c")
    s = lax.axis_index("s")
    wid = s * 2 + c
    b0 = wid * BPW
    pltpu.sync_copy(iu_hbm.at[pl.ds(b0, BPW)], b_iu)
    pltpu.sync_copy(ii_hbm.at[pl.ds(b0, BPW)], b_ii)
    pltpu.sync_copy(ij_hbm.at[pl.ds(b0, BPW)], b_ij)

    def gather_to(idxbuf, off, dstbuf):
        def addoff(i, _):
            sl = pl.ds(i * 16, 16)
            gidx[sl] = idxbuf[sl] + off
            return 0
        lax.fori_loop(0, BPW // 16, addoff, 0)
        pltpu.async_copy(ot_hbm.at[gidx], dstbuf, sem).wait()

    gather_to(b_iu, 8 * NP, ulo)
    gather_to(b_iu, 9 * NP, uhi)
    gather_to(b_ii, 8 * NP, ilo)
    gather_to(b_ii, 9 * NP, ihi)
    gather_to(b_ij, 8 * NP, jlo)
    gather_to(b_ij, 9 * NP, jhi)

    racc = jnp.zeros((16,), jnp.float32)
    for idxbuf in (b_iu, b_ii, b_ij):
        for h in (0, 1):
            gather_to(idxbuf, h * NP, g)

            def sq(r, a):
                va = g[r, pl.ds(0, 16)]
                vb = g[r, pl.ds(16, 16)]
                return a + va * va + vb * vb
            racc = lax.fori_loop(0, BPW, sq, racc)
    rbuf[pl.ds(0, 16)] = racc
    pltpu.sync_copy(rbuf, out_r.at[wid])

    lanes = lax.iota(jnp.int32, 16)

    def dot16(t, _):
        s0 = pl.ds(0, 16)
        s1 = pl.ds(16, 16)
        piv = jnp.zeros((16,), jnp.float32)
        pjv = jnp.zeros((16,), jnp.float32)
        for k in range(16):
            r = t * 16 + k
            u0 = ulo[r, s0]
            u1 = ulo[r, s1]
            u2 = uhi[r, s0]
            u3 = uhi[r, s1]
            pi = jnp.sum(u0 * ilo[r, s0] + u1 * ilo[r, s1]
                         + u2 * ihi[r, s0] + u3 * ihi[r, s1])
            pj = jnp.sum(u0 * jlo[r, s0] + u1 * jlo[r, s1]
                         + u2 * jhi[r, s0] + u3 * jhi[r, s1])
            piv = jnp.where(lanes == k, pi, piv)
            pjv = jnp.where(lanes == k, pj, pjv)
        pi_buf[pl.ds(t * 16, 16)] = piv
        pj_buf[pl.ds(t * 16, 16)] = pjv
        return 0
    lax.fori_loop(0, BPW // 16, dot16, 0)
    pltpu.sync_copy(pi_buf, out_i.at[pl.ds(b0, BPW)])
    pltpu.sync_copy(pj_buf, out_j.at[pl.ds(b0, BPW)])


_score_kernel = functools.partial(
    pl.kernel,
    out_type=(
        jax.ShapeDtypeStruct((B,), jnp.float32),
        jax.ShapeDtypeStruct((B,), jnp.float32),
        jax.ShapeDtypeStruct((32, 16), jnp.float32),
    ),
    mesh=_mesh,
    scratch_types=[
        pltpu.VMEM((BPW,), jnp.int32),
        pltpu.VMEM((BPW,), jnp.int32),
        pltpu.VMEM((BPW,), jnp.int32),
        pltpu.VMEM((BPW,), jnp.int32),
        pltpu.VMEM((BPW, H), jnp.float32),
        pltpu.VMEM((BPW, H), jnp.float32),
        pltpu.VMEM((BPW, H), jnp.float32),
        pltpu.VMEM((BPW, H), jnp.float32),
        pltpu.VMEM((BPW, H), jnp.float32),
        pltpu.VMEM((BPW, H), jnp.float32),
        pltpu.VMEM((BPW, H), jnp.float32),
        pltpu.VMEM((BPW,), jnp.float32),
        pltpu.VMEM((BPW,), jnp.float32),
        pltpu.VMEM((16,), jnp.float32),
        pltpu.SemaphoreType.DMA,
    ],
    compiler_params=_params,
)(_score_body)


def kernel(user, item_i, item_j, timestamp, split_idx,
           embed_user_0, embed_item_0, graph_src, graph_dst, graph_val):
    t0 = jnp.concatenate([embed_user_0, embed_item_0], axis=0)
    t0 = jnp.concatenate(
        [t0, jnp.zeros((NP - NN, D), jnp.float32)], axis=0)
    t0s = jnp.concatenate([t0[:, :H], t0[:, H:]], axis=0)

    epad = EPAD - NEDGES
    zi = jnp.zeros((epad,), jnp.int32)
    src = jnp.concatenate([graph_src.astype(jnp.int32), zi])
    dst = jnp.concatenate([graph_dst.astype(jnp.int32), zi])
    val = jnp.concatenate([graph_val, jnp.zeros((epad,), jnp.float32)])
    vbits = jax.lax.bitcast_convert_type(val, jnp.int32)
    pk = jnp.stack([src.reshape(16 * NU, GB),
                    dst.reshape(16 * NU, GB),
                    vbits.reshape(16 * NU, GB)], axis=1)

    ot = _prop_kernel(t0s, pk)

    iu = user.astype(jnp.int32)
    ii = item_i.astype(jnp.int32) + NUSERS
    ij = item_j.astype(jnp.int32) + NUSERS
    pred_i, pred_j, reg_parts = _score_kernel(ot, iu, ii, ij)
    reg_loss = 0.5 * jnp.sum(reg_parts) / float(B)
    return pred_i, pred_j, reg_loss
